# Initial kernel scaffold; baseline (speedup 1.0000x reference)
import jax
import jax.numpy as jnp
from jax.experimental import pallas as pl


def kernel(x, edge_index, W1, b1, fc1_W, fc1_b, fc2_W, fc2_b):
    raise NotImplementedError("write your pallas kernel here")


# speedup vs baseline: 2.5257x; 2.5257x over previous
import functools

import jax
import jax.numpy as jnp
from jax import lax
from jax.experimental import pallas as pl
from jax.experimental.pallas import tpu as pltpu
from jax.experimental.pallas import tpu_sc as plsc

N = 24
F = 512
H = 8
E = 576
G = E // 16


def _stage_a_body(x_ref, w1_ref, ei_ref, xw_ref, dinv_ref):
    xw_ref[...] = jnp.dot(x_ref[...], w1_ref[...],
                          preferred_element_type=jnp.float32)
    dst = ei_ref[1:2, :]
    rows = lax.broadcasted_iota(jnp.int32, (N, E), 0)
    onehot = (jnp.broadcast_to(dst, (N, E)) == rows).astype(jnp.float32)
    deg = jnp.sum(onehot, axis=1, keepdims=True) + 1.0
    dinv_ref[...] = lax.rsqrt(deg)


def _stage_a(x, w1, ei):
    return pl.pallas_call(
        _stage_a_body,
        out_shape=[
            jax.ShapeDtypeStruct((N, H), jnp.float32),
            jax.ShapeDtypeStruct((N, 1), jnp.float32),
        ],
    )(x, w1, ei)


def _gcn_sc_body(ei_hbm, xw_hbm, dinv_hbm, b1_hbm, out_hbm,
                 ei_v, xw_v, dinv_v, b1_v, row_v):
    t = lax.axis_index("s") * 2 + lax.axis_index("c")

    @pl.when(t < N)
    def _():
        pltpu.sync_copy(ei_hbm, ei_v)
        pltpu.sync_copy(xw_hbm, xw_v.at[pl.ds(0, N * H)])
        pltpu.sync_copy(dinv_hbm, dinv_v.at[pl.ds(0, N)])
        pltpu.sync_copy(b1_hbm, b1_v.at[pl.ds(0, H)])

        tvec = jnp.full((16,), t, jnp.int32)
        dinv_t = plsc.load_gather(dinv_v, [tvec])

        def body(g, accs):
            base = g * 16
            s16 = ei_v[0, pl.ds(base, 16)]
            d16 = ei_v[1, pl.ds(base, 16)]
            w = plsc.load_gather(dinv_v, [s16]) * dinv_t
            w = jnp.where(d16 == t, w, 0.0)
            s16h = s16 * H
            return tuple(
                accs[h] + w * plsc.load_gather(xw_v, [s16h + h])
                for h in range(H))

        accs = lax.fori_loop(
            0, G, body, tuple(jnp.zeros((16,), jnp.float32) for _ in range(H)))

        iota = lax.iota(jnp.int32, 16)
        row = jnp.zeros((16,), jnp.float32)
        for h in range(H):
            row = jnp.where(iota == h, jnp.sum(accs[h]), row)
        xw_t = plsc.load_gather(xw_v, [tvec * H + (iota & (H - 1))])
        row = row + dinv_t * dinv_t * xw_t + b1_v[...]
        row = jnp.where(iota < H, jnp.maximum(row, 0.0), 0.0)
        row_v[...] = row
        pltpu.sync_copy(row_v.at[pl.ds(0, H)], out_hbm.at[pl.ds(t * H, H)])


def _gcn_sc(ei, xw, dinv, b1):
    mesh = plsc.VectorSubcoreMesh(core_axis_name="c", subcore_axis_name="s",
                                  num_cores=2, num_subcores=16)
    return pl.kernel(
        _gcn_sc_body,
        out_type=jax.ShapeDtypeStruct((N * H,), jnp.float32),
        mesh=mesh,
        compiler_params=pltpu.CompilerParams(needs_layout_passes=False),
        scratch_types=[
            pltpu.VMEM((2, E), jnp.int32),
            pltpu.VMEM((256,), jnp.float32),
            pltpu.VMEM((128,), jnp.float32),
            pltpu.VMEM((16,), jnp.float32),
            pltpu.VMEM((16,), jnp.float32),
        ],
    )(ei, xw, dinv, b1)


def _fc_body(hv_ref, w1_ref, b1_ref, w2_ref, b2_ref, out_ref):
    v = hv_ref[...]
    o1 = lax.dot_general(v, w1_ref[...], (((1,), (1,)), ((), ())),
                         preferred_element_type=jnp.float32) + b1_ref[...]
    o2 = lax.dot_general(o1, w2_ref[...], (((1,), (1,)), ((), ())),
                         preferred_element_type=jnp.float32) + b2_ref[...]
    m = jnp.max(o2, axis=1, keepdims=True)
    e = o2 - m
    out_ref[...] = e - jnp.log(jnp.sum(jnp.exp(e), axis=1, keepdims=True))


def _fc(hv, fc1_w, fc1_b, fc2_w, fc2_b):
    return pl.pallas_call(
        _fc_body,
        out_shape=jax.ShapeDtypeStruct((1, 2), jnp.float32),
    )(hv, fc1_w, fc1_b, fc2_w, fc2_b)


def kernel(x, edge_index, W1, b1, fc1_W, fc1_b, fc2_W, fc2_b):
    xw, dinv = _stage_a(x, W1, edge_index)
    h = _gcn_sc(edge_index, xw.reshape(N * H), dinv.reshape(N), b1)
    return _fc(h.reshape(1, N * H), fc1_W, fc1_b.reshape(1, 128),
               fc2_W, fc2_b.reshape(1, 2))
